# baseline (speedup 1.0000x reference)
import jax
import jax.numpy as jnp
from jax.experimental import pallas as pl

_BM = 400


def _layer_body(a_ref, x_ref, b_ref, y_ref):
    y = jnp.dot(a_ref[...], x_ref[...], preferred_element_type=jnp.float32)
    y_ref[...] = y + b_ref[...]


def _final_body(a_ref, x2_ref, fea_ref, x1_ref, b_ref, out_ref):
    i = pl.program_id(0)
    y3 = jnp.dot(a_ref[...], x2_ref[...], preferred_element_type=jnp.float32)
    x2_blk = x2_ref[pl.ds(i * _BM, _BM), :]
    out_ref[...] = (fea_ref[...] + x1_ref[...] + x2_blk + y3 + b_ref[...]) * 0.25


def _spmm_bias(adj, x, b):
    n, d = x.shape
    grid = (n // _BM,)
    return pl.pallas_call(
        _layer_body,
        grid=grid,
        in_specs=[
            pl.BlockSpec((_BM, n), lambda i: (i, 0)),
            pl.BlockSpec((n, d), lambda i: (0, 0)),
            pl.BlockSpec((1, d), lambda i: (0, 0)),
        ],
        out_specs=pl.BlockSpec((_BM, d), lambda i: (i, 0)),
        out_shape=jax.ShapeDtypeStruct((n, d), jnp.float32),
    )(adj, x, b)


def _final_layer(adj, x2, fea, x1, b):
    n, d = fea.shape
    grid = (n // _BM,)
    return pl.pallas_call(
        _final_body,
        grid=grid,
        in_specs=[
            pl.BlockSpec((_BM, n), lambda i: (i, 0)),
            pl.BlockSpec((n, d), lambda i: (0, 0)),
            pl.BlockSpec((_BM, d), lambda i: (i, 0)),
            pl.BlockSpec((_BM, d), lambda i: (i, 0)),
            pl.BlockSpec((1, d), lambda i: (0, 0)),
        ],
        out_specs=pl.BlockSpec((_BM, d), lambda i: (i, 0)),
        out_shape=jax.ShapeDtypeStruct((n, d), jnp.float32),
    )(adj, x2, fea, x1, b)


def kernel(fea, adj, b0, b1, b2):
    d = fea.shape[1]
    x1 = _spmm_bias(adj, fea, b0.reshape(1, d))
    x2 = _spmm_bias(adj, x1, b1.reshape(1, d))
    return _final_layer(adj, x2, fea, x1, b2.reshape(1, d))


# speedup vs baseline: 1.4069x; 1.4069x over previous
import jax
import jax.numpy as jnp
from jax.experimental import pallas as pl

_BM1 = 256
_BM2 = 512
_F8 = jnp.float8_e4m3fn


def _l1_body(a_ref, x_ref, b_ref, y_ref, aq_ref, rs_ref):
    a = a_ref[...]
    aq_ref[...] = a.astype(_F8)
    rs_ref[...] = jnp.sum(a, axis=1, keepdims=True)
    y = jnp.dot(a, x_ref[...], preferred_element_type=jnp.float32)
    y_ref[...] = y + b_ref[...]


def _layer1(adj, fea, b, npad):
    n, d = fea.shape
    grid = (npad // _BM1,)
    return pl.pallas_call(
        _l1_body,
        grid=grid,
        in_specs=[
            pl.BlockSpec((_BM1, n), lambda i: (i, 0)),
            pl.BlockSpec((n, d), lambda i: (0, 0)),
            pl.BlockSpec((1, d), lambda i: (0, 0)),
        ],
        out_specs=[
            pl.BlockSpec((_BM1, d), lambda i: (i, 0)),
            pl.BlockSpec((_BM1, n), lambda i: (i, 0)),
            pl.BlockSpec((_BM1, 1), lambda i: (i, 0)),
        ],
        out_shape=[
            jax.ShapeDtypeStruct((n, d), jnp.float32),
            jax.ShapeDtypeStruct((npad, n), _F8),
            jax.ShapeDtypeStruct((npad, 1), jnp.float32),
        ],
    )(adj, fea, b)


def _l2_body(aq_ref, xq_ref, rs_ref, s_ref, m_ref, b_ref, y_ref):
    y = jnp.dot(aq_ref[...], xq_ref[...], preferred_element_type=jnp.float32)
    y_ref[...] = y * s_ref[...] + rs_ref[...] * m_ref[...] + b_ref[...]


def _layer2(adj_q, x_q, rs, s_vec, m_vec, b, n, d):
    grid = (adj_q.shape[0] // _BM2,)
    return pl.pallas_call(
        _l2_body,
        grid=grid,
        in_specs=[
            pl.BlockSpec((_BM2, n), lambda i: (i, 0)),
            pl.BlockSpec((n, d), lambda i: (0, 0)),
            pl.BlockSpec((_BM2, 1), lambda i: (i, 0)),
            pl.BlockSpec((1, d), lambda i: (0, 0)),
            pl.BlockSpec((1, d), lambda i: (0, 0)),
            pl.BlockSpec((1, d), lambda i: (0, 0)),
        ],
        out_specs=pl.BlockSpec((_BM2, d), lambda i: (i, 0)),
        out_shape=jax.ShapeDtypeStruct((n, d), jnp.float32),
    )(adj_q, x_q, rs, s_vec, m_vec, b)


def _l3_body(aq_ref, xq_ref, rs_ref, s_ref, m_ref, b_ref, fea_ref, x1_ref, x2_ref,
             out_ref):
    y = jnp.dot(aq_ref[...], xq_ref[...], preferred_element_type=jnp.float32)
    y = y * s_ref[...] + rs_ref[...] * m_ref[...] + b_ref[...]
    out_ref[...] = (fea_ref[...] + x1_ref[...] + x2_ref[...] + y) * 0.25


def _layer3(adj_q, x_q, rs, s_vec, m_vec, b, fea, x1, x2):
    n, d = fea.shape
    grid = (adj_q.shape[0] // _BM2,)
    return pl.pallas_call(
        _l3_body,
        grid=grid,
        in_specs=[
            pl.BlockSpec((_BM2, n), lambda i: (i, 0)),
            pl.BlockSpec((n, d), lambda i: (0, 0)),
            pl.BlockSpec((_BM2, 1), lambda i: (i, 0)),
            pl.BlockSpec((1, d), lambda i: (0, 0)),
            pl.BlockSpec((1, d), lambda i: (0, 0)),
            pl.BlockSpec((1, d), lambda i: (0, 0)),
            pl.BlockSpec((_BM2, d), lambda i: (i, 0)),
            pl.BlockSpec((_BM2, d), lambda i: (i, 0)),
            pl.BlockSpec((_BM2, d), lambda i: (i, 0)),
        ],
        out_specs=pl.BlockSpec((_BM2, d), lambda i: (i, 0)),
        out_shape=jax.ShapeDtypeStruct((n, d), jnp.float32),
    )(adj_q, x_q, rs, s_vec, m_vec, b, fea, x1, x2)


def _quantize(x, d):
    m = jnp.mean(x, axis=0, keepdims=True)
    xc = x - m
    s = jnp.maximum(jnp.max(jnp.abs(xc)), 1e-30) * (1.0 / 384.0)
    x_q = (xc * (1.0 / s)).astype(_F8)
    return x_q, jnp.full((1, d), s, jnp.float32), m


def kernel(fea, adj, b0, b1, b2):
    n, d = fea.shape
    npad = -(-n // _BM2) * _BM2
    x1, adj_q, rs = _layer1(adj, fea, b0.reshape(1, d), npad)
    x1q, s1v, m1 = _quantize(x1, d)
    x2 = _layer2(adj_q, x1q, rs, s1v, m1, b1.reshape(1, d), n, d)
    x2q, s2v, m2 = _quantize(x2, d)
    return _layer3(adj_q, x2q, rs, s2v, m2, b2.reshape(1, d), fea, x1, x2)


# speedup vs baseline: 1.4311x; 1.0172x over previous
import functools

import jax
import jax.numpy as jnp
from jax import lax
from jax.experimental import pallas as pl

_BM1 = 256
_BM2 = 512
_F8 = jnp.float8_e4m3fn
_BF16 = jnp.bfloat16


def _col_stats(y, valid_rows, cs_ref, mx_ref, mn_ref, first):
    rows = lax.broadcasted_iota(jnp.int32, (y.shape[0], 1), 0)
    ok = rows < valid_rows
    cs = jnp.sum(jnp.where(ok, y, 0.0), axis=0, keepdims=True)
    mx = jnp.max(jnp.where(ok, y, -jnp.inf), axis=0, keepdims=True)
    mn = jnp.min(jnp.where(ok, y, jnp.inf), axis=0, keepdims=True)

    @pl.when(first)
    def _():
        cs_ref[...] = cs
        mx_ref[...] = mx
        mn_ref[...] = mn

    @pl.when(jnp.logical_not(first))
    def _():
        cs_ref[...] += cs
        mx_ref[...] = jnp.maximum(mx_ref[...], mx)
        mn_ref[...] = jnp.minimum(mn_ref[...], mn)


def _l1_body(a_ref, x_ref, b_ref, y_ref, aq_ref, rs_ref, cs_ref, mx_ref,
             mn_ref, *, n):
    i = pl.program_id(0)
    a = a_ref[...]
    aq_ref[...] = a.astype(_F8)
    rs_ref[...] = jnp.sum(a, axis=1, keepdims=True)
    y = jnp.dot(a, x_ref[...], preferred_element_type=jnp.float32)
    y = y + b_ref[...]
    y_ref[...] = y.astype(_BF16)
    _col_stats(y, n - i * _BM1, cs_ref, mx_ref, mn_ref, i == 0)


def _layer1(adj, fea, b, npad):
    n, d = fea.shape
    grid = (npad // _BM1,)
    one = lambda i: (0, 0)
    return pl.pallas_call(
        functools.partial(_l1_body, n=n),
        grid=grid,
        in_specs=[
            pl.BlockSpec((_BM1, n), lambda i: (i, 0)),
            pl.BlockSpec((n, d), one),
            pl.BlockSpec((1, d), one),
        ],
        out_specs=[
            pl.BlockSpec((_BM1, d), lambda i: (i, 0)),
            pl.BlockSpec((_BM1, n), lambda i: (i, 0)),
            pl.BlockSpec((_BM1, 1), lambda i: (i, 0)),
            pl.BlockSpec((1, d), one),
            pl.BlockSpec((1, d), one),
            pl.BlockSpec((1, d), one),
        ],
        out_shape=[
            jax.ShapeDtypeStruct((n, d), _BF16),
            jax.ShapeDtypeStruct((npad, n), _F8),
            jax.ShapeDtypeStruct((npad, 1), jnp.float32),
            jax.ShapeDtypeStruct((1, d), jnp.float32),
            jax.ShapeDtypeStruct((1, d), jnp.float32),
            jax.ShapeDtypeStruct((1, d), jnp.float32),
        ],
    )(adj, fea, b)


def _l2_body(aq_ref, xq_ref, rs_ref, s_ref, m_ref, b_ref, y_ref, cs_ref,
             mx_ref, mn_ref, *, n):
    i = pl.program_id(0)
    y = jnp.dot(aq_ref[...], xq_ref[...], preferred_element_type=jnp.float32)
    y = y * s_ref[...] + rs_ref[...] * m_ref[...] + b_ref[...]
    y_ref[...] = y.astype(_BF16)
    _col_stats(y, n - i * _BM2, cs_ref, mx_ref, mn_ref, i == 0)


def _layer2(adj_q, x_q, rs, s_vec, m_vec, b, n, d):
    grid = (adj_q.shape[0] // _BM2,)
    one = lambda i: (0, 0)
    return pl.pallas_call(
        functools.partial(_l2_body, n=n),
        grid=grid,
        in_specs=[
            pl.BlockSpec((_BM2, n), lambda i: (i, 0)),
            pl.BlockSpec((n, d), one),
            pl.BlockSpec((_BM2, 1), lambda i: (i, 0)),
            pl.BlockSpec((1, d), one),
            pl.BlockSpec((1, d), one),
            pl.BlockSpec((1, d), one),
        ],
        out_specs=[
            pl.BlockSpec((_BM2, d), lambda i: (i, 0)),
            pl.BlockSpec((1, d), one),
            pl.BlockSpec((1, d), one),
            pl.BlockSpec((1, d), one),
        ],
        out_shape=[
            jax.ShapeDtypeStruct((n, d), _BF16),
            jax.ShapeDtypeStruct((1, d), jnp.float32),
            jax.ShapeDtypeStruct((1, d), jnp.float32),
            jax.ShapeDtypeStruct((1, d), jnp.float32),
        ],
    )(adj_q, x_q, rs, s_vec, m_vec, b)


def _l3_body(aq_ref, xq_ref, rs_ref, s2_ref, m2_ref, b_ref, fea_ref, x1_ref,
             out_ref):
    i = pl.program_id(0)
    y = jnp.dot(aq_ref[...], xq_ref[...], preferred_element_type=jnp.float32)
    y = y * s2_ref[...] + rs_ref[...] * m2_ref[...] + b_ref[...]
    x2_blk = xq_ref[pl.ds(i * _BM2, _BM2), :].astype(jnp.float32)
    x2_blk = x2_blk * s2_ref[...] + m2_ref[...]
    x1_blk = x1_ref[...].astype(jnp.float32)
    out_ref[...] = (fea_ref[...] + x1_blk + x2_blk + y) * 0.25


def _layer3(adj_q, x_q, rs, s2_vec, m2_vec, b, fea, x1):
    n, d = fea.shape
    grid = (adj_q.shape[0] // _BM2,)
    one = lambda i: (0, 0)
    return pl.pallas_call(
        _l3_body,
        grid=grid,
        in_specs=[
            pl.BlockSpec((_BM2, n), lambda i: (i, 0)),
            pl.BlockSpec((n, d), one),
            pl.BlockSpec((_BM2, 1), lambda i: (i, 0)),
            pl.BlockSpec((1, d), one),
            pl.BlockSpec((1, d), one),
            pl.BlockSpec((1, d), one),
            pl.BlockSpec((_BM2, d), lambda i: (i, 0)),
            pl.BlockSpec((_BM2, d), lambda i: (i, 0)),
        ],
        out_specs=pl.BlockSpec((_BM2, d), lambda i: (i, 0)),
        out_shape=jax.ShapeDtypeStruct((n, d), jnp.float32),
    )(adj_q, x_q, rs, s2_vec, m2_vec, b, fea, x1)


def _quantize(x, colsum, colmax, colmin, n, d):
    m = colsum * (1.0 / n)
    s = jnp.max(jnp.maximum(colmax - m, m - colmin)) * (1.0 / 384.0)
    s = jnp.maximum(s, 1e-30)
    x_q = ((x.astype(jnp.float32) - m) * (1.0 / s)).astype(_F8)
    return x_q, jnp.full((1, d), s, jnp.float32), m


def kernel(fea, adj, b0, b1, b2):
    n, d = fea.shape
    npad = -(-n // _BM2) * _BM2
    x1, adj_q, rs, cs1, mx1, mn1 = _layer1(adj, fea, b0.reshape(1, d), npad)
    x1q, s1v, m1 = _quantize(x1, cs1, mx1, mn1, n, d)
    x2, cs2, mx2, mn2 = _layer2(adj_q, x1q, rs, s1v, m1, b1.reshape(1, d), n, d)
    x2q, s2v, m2 = _quantize(x2, cs2, mx2, mn2, n, d)
    return _layer3(adj_q, x2q, rs, s2v, m2, b2.reshape(1, d), fea, x1)


# speedup vs baseline: 1.4910x; 1.0419x over previous
import functools
import math

import jax
import jax.numpy as jnp
from jax import lax
from jax.experimental import pallas as pl

_BM1 = 320
_BM2 = 1024
_F8 = jnp.float8_e4m3fn
_BF16 = jnp.bfloat16


def _col_stats(y, valid_rows, cs_ref, mx_ref, mn_ref, first):
    rows = lax.broadcasted_iota(jnp.int32, (y.shape[0], 1), 0)
    ok = rows < valid_rows
    cs = jnp.sum(jnp.where(ok, y, 0.0), axis=0, keepdims=True)
    mx = jnp.max(jnp.where(ok, y, -jnp.inf), axis=0, keepdims=True)
    mn = jnp.min(jnp.where(ok, y, jnp.inf), axis=0, keepdims=True)

    @pl.when(first)
    def _():
        cs_ref[...] = cs
        mx_ref[...] = mx
        mn_ref[...] = mn

    @pl.when(jnp.logical_not(first))
    def _():
        cs_ref[...] += cs
        mx_ref[...] = jnp.maximum(mx_ref[...], mx)
        mn_ref[...] = jnp.minimum(mn_ref[...], mn)


def _l1_body(a_ref, x_ref, b_ref, y_ref, aq_ref, rs_ref, cs_ref, mx_ref,
             mn_ref, *, n):
    i = pl.program_id(0)
    a = a_ref[...]
    aq_ref[...] = a.astype(_F8)
    rs_ref[...] = jnp.sum(a, axis=1, keepdims=True)
    y = jnp.dot(a, x_ref[...], preferred_element_type=jnp.float32)
    y = y + b_ref[...]
    y_ref[...] = y.astype(_BF16)
    _col_stats(y, n - i * _BM1, cs_ref, mx_ref, mn_ref, i == 0)


def _layer1(adj, fea, b, npad):
    n, d = fea.shape
    grid = (npad // _BM1,)
    one = lambda i: (0, 0)
    return pl.pallas_call(
        functools.partial(_l1_body, n=n),
        grid=grid,
        in_specs=[
            pl.BlockSpec((_BM1, n), lambda i: (i, 0)),
            pl.BlockSpec((n, d), one),
            pl.BlockSpec((1, d), one),
        ],
        out_specs=[
            pl.BlockSpec((_BM1, d), lambda i: (i, 0)),
            pl.BlockSpec((_BM1, n), lambda i: (i, 0)),
            pl.BlockSpec((_BM1, 1), lambda i: (i, 0)),
            pl.BlockSpec((1, d), one),
            pl.BlockSpec((1, d), one),
            pl.BlockSpec((1, d), one),
        ],
        out_shape=[
            jax.ShapeDtypeStruct((n, d), _BF16),
            jax.ShapeDtypeStruct((npad, n), _F8),
            jax.ShapeDtypeStruct((npad, 1), jnp.float32),
            jax.ShapeDtypeStruct((1, d), jnp.float32),
            jax.ShapeDtypeStruct((1, d), jnp.float32),
            jax.ShapeDtypeStruct((1, d), jnp.float32),
        ],
    )(adj, fea, b)


def _l2_body(aq_ref, xq_ref, rs_ref, s_ref, m_ref, b_ref, y_ref, cs_ref,
             mx_ref, mn_ref, *, n):
    i = pl.program_id(0)
    y = jnp.dot(aq_ref[...], xq_ref[...], preferred_element_type=jnp.float32)
    y = y * s_ref[...] + rs_ref[...] * m_ref[...] + b_ref[...]
    y_ref[...] = y.astype(_BF16)
    _col_stats(y, n - i * _BM2, cs_ref, mx_ref, mn_ref, i == 0)


def _layer2(adj_q, x_q, rs, s_vec, m_vec, b, n, d):
    grid = (adj_q.shape[0] // _BM2,)
    one = lambda i: (0, 0)
    return pl.pallas_call(
        functools.partial(_l2_body, n=n),
        grid=grid,
        in_specs=[
            pl.BlockSpec((_BM2, n), lambda i: (i, 0)),
            pl.BlockSpec((n, d), one),
            pl.BlockSpec((_BM2, 1), lambda i: (i, 0)),
            pl.BlockSpec((1, d), one),
            pl.BlockSpec((1, d), one),
            pl.BlockSpec((1, d), one),
        ],
        out_specs=[
            pl.BlockSpec((_BM2, d), lambda i: (i, 0)),
            pl.BlockSpec((1, d), one),
            pl.BlockSpec((1, d), one),
            pl.BlockSpec((1, d), one),
        ],
        out_shape=[
            jax.ShapeDtypeStruct((n, d), _BF16),
            jax.ShapeDtypeStruct((1, d), jnp.float32),
            jax.ShapeDtypeStruct((1, d), jnp.float32),
            jax.ShapeDtypeStruct((1, d), jnp.float32),
        ],
    )(adj_q, x_q, rs, s_vec, m_vec, b)


def _l3_body(aq_ref, xq_ref, rs_ref, s2_ref, m2_ref, b_ref, fea_ref, x1_ref,
             out_ref):
    i = pl.program_id(0)
    y = jnp.dot(aq_ref[...], xq_ref[...], preferred_element_type=jnp.float32)
    y = y * s2_ref[...] + rs_ref[...] * m2_ref[...] + b_ref[...]
    x2_blk = xq_ref[pl.ds(i * _BM2, _BM2), :].astype(jnp.float32)
    x2_blk = x2_blk * s2_ref[...] + m2_ref[...]
    x1_blk = x1_ref[...].astype(jnp.float32)
    out_ref[...] = (fea_ref[...] + x1_blk + x2_blk + y) * 0.25


def _layer3(adj_q, x_q, rs, s2_vec, m2_vec, b, fea, x1):
    n, d = fea.shape
    grid = (adj_q.shape[0] // _BM2,)
    one = lambda i: (0, 0)
    return pl.pallas_call(
        _l3_body,
        grid=grid,
        in_specs=[
            pl.BlockSpec((_BM2, n), lambda i: (i, 0)),
            pl.BlockSpec((n, d), one),
            pl.BlockSpec((_BM2, 1), lambda i: (i, 0)),
            pl.BlockSpec((1, d), one),
            pl.BlockSpec((1, d), one),
            pl.BlockSpec((1, d), one),
            pl.BlockSpec((_BM2, d), lambda i: (i, 0)),
            pl.BlockSpec((_BM2, d), lambda i: (i, 0)),
        ],
        out_specs=pl.BlockSpec((_BM2, d), lambda i: (i, 0)),
        out_shape=jax.ShapeDtypeStruct((n, d), jnp.float32),
    )(adj_q, x_q, rs, s2_vec, m2_vec, b, fea, x1)


def _quantize(x, colsum, colmax, colmin, n, d):
    m = colsum * (1.0 / n)
    s = jnp.max(jnp.maximum(colmax - m, m - colmin)) * (1.0 / 384.0)
    s = jnp.maximum(s, 1e-30)
    x_q = ((x.astype(jnp.float32) - m) * (1.0 / s)).astype(_F8)
    return x_q, jnp.full((1, d), s, jnp.float32), m


def kernel(fea, adj, b0, b1, b2):
    n, d = fea.shape
    blk = math.lcm(_BM1, _BM2)
    npad = -(-n // blk) * blk
    x1, adj_q, rs, cs1, mx1, mn1 = _layer1(adj, fea, b0.reshape(1, d), npad)
    x1q, s1v, m1 = _quantize(x1, cs1, mx1, mn1, n, d)
    x2, cs2, mx2, mn2 = _layer2(adj_q, x1q, rs, s1v, m1, b1.reshape(1, d), n, d)
    x2q, s2v, m2 = _quantize(x2, cs2, mx2, mn2, n, d)
    return _layer3(adj_q, x2q, rs, s2v, m2, b2.reshape(1, d), fea, x1)


# speedup vs baseline: 1.6193x; 1.0860x over previous
import functools
import math

import jax
import jax.numpy as jnp
from jax import lax
from jax.experimental import pallas as pl
from jax.experimental.pallas import tpu as pltpu

_BM1 = 320
_BM2 = 1024
_F8 = jnp.float8_e4m3fn
_BF16 = jnp.bfloat16


def _col_stats(y, valid_rows, cs_ref, mx_ref, mn_ref, first):
    rows = lax.broadcasted_iota(jnp.int32, (y.shape[0], 1), 0)
    ok = rows < valid_rows
    cs = jnp.sum(jnp.where(ok, y, 0.0), axis=0, keepdims=True)
    mx = jnp.max(jnp.where(ok, y, -jnp.inf), axis=0, keepdims=True)
    mn = jnp.min(jnp.where(ok, y, jnp.inf), axis=0, keepdims=True)

    @pl.when(first)
    def _():
        cs_ref[...] = cs
        mx_ref[...] = mx
        mn_ref[...] = mn

    @pl.when(jnp.logical_not(first))
    def _():
        cs_ref[...] += cs
        mx_ref[...] = jnp.maximum(mx_ref[...], mx)
        mn_ref[...] = jnp.minimum(mn_ref[...], mn)


def _l1_body(a_ref, x_ref, b_ref, y_ref, aq_ref, rs_ref, cs_ref, mx_ref,
             mn_ref, *, n):
    i = pl.program_id(0)
    a = a_ref[...]
    aq_ref[...] = a.astype(_F8)
    rs_ref[...] = jnp.sum(a, axis=1, keepdims=True)
    y = jnp.dot(a, x_ref[...], preferred_element_type=jnp.float32)
    y = y + b_ref[...]
    y_ref[...] = y.astype(_BF16)
    _col_stats(y, n - i * _BM1, cs_ref, mx_ref, mn_ref, i == 0)


def _layer1(adj, fea, b, npad):
    n, d = fea.shape
    grid = (npad // _BM1,)
    one = lambda i: (0, 0)
    return pl.pallas_call(
        functools.partial(_l1_body, n=n),
        grid=grid,
        in_specs=[
            pl.BlockSpec((_BM1, n), lambda i: (i, 0)),
            pl.BlockSpec((n, d), one),
            pl.BlockSpec((1, d), one),
        ],
        out_specs=[
            pl.BlockSpec((_BM1, d), lambda i: (i, 0)),
            pl.BlockSpec((_BM1, n), lambda i: (i, 0)),
            pl.BlockSpec((_BM1, 1), lambda i: (i, 0)),
            pl.BlockSpec((1, d), one),
            pl.BlockSpec((1, d), one),
            pl.BlockSpec((1, d), one),
        ],
        out_shape=[
            jax.ShapeDtypeStruct((n, d), _BF16),
            jax.ShapeDtypeStruct((npad, n), _F8),
            jax.ShapeDtypeStruct((npad, 1), jnp.float32),
            jax.ShapeDtypeStruct((1, d), jnp.float32),
            jax.ShapeDtypeStruct((1, d), jnp.float32),
            jax.ShapeDtypeStruct((1, d), jnp.float32),
        ],
    )(adj, fea, b)


def _l23_body(aq_ref, x1_ref, rs_ref, cs1_ref, mx1_ref, mn1_ref, b1_ref,
              b2_ref, fea_ref, x1b_ref, out_ref, x1q_s, x2_s, x2q_s, cs2_s,
              mx2_s, mn2_s, *, n):
    p = pl.program_id(0)
    i = pl.program_id(1)
    inv_n = 1.0 / n

    def scale_of(cs, mx, mn):
        m = cs * inv_n
        s = jnp.max(jnp.maximum(mx - m, m - mn)) * (1.0 / 384.0)
        return m, jnp.maximum(s, 1e-30)

    m1, s1 = scale_of(cs1_ref[...], mx1_ref[...], mn1_ref[...])

    @pl.when((p == 0) & (i == 0))
    def _():
        x1q_s[...] = ((x1_ref[...].astype(jnp.float32) - m1) *
                      (1.0 / s1)).astype(_F8)

    @pl.when(p == 0)
    def _():
        y = jnp.dot(aq_ref[...], x1q_s[...],
                    preferred_element_type=jnp.float32)
        y = y * s1 + rs_ref[...] * m1 + b1_ref[...]
        x2_s[pl.ds(i * _BM2, _BM2), :] = y.astype(_BF16)
        _col_stats(y, n - i * _BM2, cs2_s, mx2_s, mn2_s, i == 0)

    m2, s2 = scale_of(cs2_s[...], mx2_s[...], mn2_s[...])

    @pl.when((p == 1) & (i == 0))
    def _():
        x2q_s[...] = ((x2_s[pl.ds(0, n), :].astype(jnp.float32) - m2) *
                      (1.0 / s2)).astype(_F8)

    @pl.when(p == 1)
    def _():
        y = jnp.dot(aq_ref[...], x2q_s[...],
                    preferred_element_type=jnp.float32)
        y = y * s2 + rs_ref[...] * m2 + b2_ref[...]
        x1_blk = x1b_ref[...].astype(jnp.float32)
        x2_blk = x2_s[pl.ds(i * _BM2, _BM2), :].astype(jnp.float32)
        out_ref[...] = (fea_ref[...] + x1_blk + x2_blk + y) * 0.25


def _layers23(adj_q, x1, rs, cs1, mx1, mn1, b1, b2, fea):
    n, d = fea.shape
    npad = adj_q.shape[0]
    grid = (2, npad // _BM2)
    one = lambda p, i: (0, 0)
    ph1 = lambda p, i: (i * p, 0)
    return pl.pallas_call(
        functools.partial(_l23_body, n=n),
        grid=grid,
        in_specs=[
            pl.BlockSpec((_BM2, n), lambda p, i: (i, 0)),
            pl.BlockSpec((n, d), one),
            pl.BlockSpec((_BM2, 1), lambda p, i: (i, 0)),
            pl.BlockSpec((1, d), one),
            pl.BlockSpec((1, d), one),
            pl.BlockSpec((1, d), one),
            pl.BlockSpec((1, d), one),
            pl.BlockSpec((1, d), one),
            pl.BlockSpec((_BM2, d), ph1),
            pl.BlockSpec((_BM2, d), ph1),
        ],
        out_specs=pl.BlockSpec((_BM2, d), ph1),
        out_shape=jax.ShapeDtypeStruct((n, d), jnp.float32),
        scratch_shapes=[
            pltpu.VMEM((n, d), _F8),
            pltpu.VMEM((npad, d), _BF16),
            pltpu.VMEM((n, d), _F8),
            pltpu.VMEM((1, d), jnp.float32),
            pltpu.VMEM((1, d), jnp.float32),
            pltpu.VMEM((1, d), jnp.float32),
        ],
    )(adj_q, x1, rs, cs1, mx1, mn1, b1, b2, fea, x1)


def kernel(fea, adj, b0, b1, b2):
    n, d = fea.shape
    blk = math.lcm(_BM1, _BM2)
    npad = -(-n // blk) * blk
    x1, adj_q, rs, cs1, mx1, mn1 = _layer1(adj, fea, b0.reshape(1, d), npad)
    return _layers23(adj_q, x1, rs, cs1, mx1, mn1, b1.reshape(1, d),
                     b2.reshape(1, d), fea)
